# Initial kernel scaffold; baseline (speedup 1.0000x reference)
import jax
import jax.numpy as jnp
from jax.experimental import pallas as pl


def kernel(hidden, mask, target_label, conv_w, conv_b, lin_w, lin_b):
    raise NotImplementedError("write your pallas kernel here")


# speedup vs baseline: 29.5210x; 29.5210x over previous
import jax
import jax.numpy as jnp
from jax.experimental import pallas as pl


def _alpha_kernel(h_ref, w0_ref, w1_ref, w2_ref, cb_ref, lw_ref, lb_ref,
                  m_ref, araw_ref):
    H = h_ref[0]
    T, D = H.shape
    p0 = jnp.dot(H, w0_ref[...], preferred_element_type=jnp.float32)
    p1 = jnp.dot(H, w1_ref[...], preferred_element_type=jnp.float32)
    p2 = jnp.dot(H, w2_ref[...], preferred_element_type=jnp.float32)
    z = jnp.zeros((1, D), jnp.float32)
    o = jnp.concatenate([z, p0[:-1]], 0) + p1 + jnp.concatenate([p2[1:], z], 0)
    o = jnp.maximum(o + cb_ref[...][0][None, :], 0.0)
    y = jnp.dot(o, lw_ref[...].reshape(-1, 1),
                preferred_element_type=jnp.float32)[:, 0] + lb_ref[0, 0]
    araw_ref[0, 0, :] = jax.nn.sigmoid(y) * m_ref[0, 0, :]


def _scan_kernel(arawT_ref, scale_ref, alphasT_ref, firesT_ref, uT_ref,
                 vT_ref, kAT_ref, nf_ref):
    T, B = arawT_ref.shape
    scale = scale_ref[0, :]

    def body(t, carry):
        integ, k = carry
        a = arawT_ref[t, :] * scale
        alphasT_ref[t, :] = a
        dist = 1.0 - integ
        integ = integ + a
        firesT_ref[t, :] = integ
        fp = integ >= 1.0
        cur = jnp.where(fp, dist, a)
        uT_ref[t, :] = cur
        vT_ref[t, :] = a - cur
        kAT_ref[t, :] = k
        integ = jnp.where(fp, integ - 1.0, integ)
        k = k + fp.astype(jnp.int32)
        return integ, k

    _, k = jax.lax.fori_loop(
        0, T, body,
        (jnp.zeros((B,), jnp.float32), jnp.zeros((B,), jnp.int32)))
    nf_ref[0, :] = k


def _embed_kernel(h_ref, u_ref, v_ref, kA_ref, rm_ref, acc_ref):
    nt = pl.program_id(1)
    L = acc_ref.shape[1]
    TT = h_ref.shape[1]
    j = jax.lax.broadcasted_iota(jnp.int32, (L, TT), 0)
    kA = kA_ref[0, 0, :][None, :]
    M = jnp.where(j == kA, u_ref[0, 0, :][None, :], 0.0)
    M = M + jnp.where(j == kA + 1, v_ref[0, 0, :][None, :], 0.0)
    M = M * rm_ref[0, 0, :][:, None]
    H = h_ref[0]
    M_hi = M.astype(jnp.bfloat16).astype(jnp.float32)
    M_lo = M - M_hi
    H_hi = H.astype(jnp.bfloat16).astype(jnp.float32)
    H_lo = H - H_hi
    contrib = (jnp.dot(M_hi, H_hi, preferred_element_type=jnp.float32)
               + jnp.dot(M_hi, H_lo, preferred_element_type=jnp.float32)
               + jnp.dot(M_lo, H_hi, preferred_element_type=jnp.float32)
               + jnp.dot(M_lo, H_lo, preferred_element_type=jnp.float32))

    @pl.when(nt == 0)
    def _():
        acc_ref[0] = contrib

    @pl.when(nt != 0)
    def _():
        acc_ref[0] = acc_ref[0] + contrib


def kernel(hidden, mask, target_label, conv_w, conv_b, lin_w, lin_b):
    B, T, D = hidden.shape
    L = target_label.shape[1]
    w0 = conv_w[:, :, 0].T
    w1 = conv_w[:, :, 1].T
    w2 = conv_w[:, :, 2].T
    cb2 = conv_b.reshape(1, D)
    lw2 = lin_w.reshape(1, D)
    lb2 = lin_b.reshape(1, 1)
    mask3 = mask.reshape(B, 1, T).astype(jnp.float32)
    full = lambda shp: pl.BlockSpec(shp, lambda b: tuple(0 for _ in shp))

    araw = pl.pallas_call(
        _alpha_kernel,
        grid=(B,),
        in_specs=[
            pl.BlockSpec((1, T, D), lambda b: (b, 0, 0)),
            full((D, D)), full((D, D)), full((D, D)),
            full((1, D)), full((1, D)), full((1, 1)),
            pl.BlockSpec((1, 1, T), lambda b: (b, 0, 0)),
        ],
        out_specs=pl.BlockSpec((1, 1, T), lambda b: (b, 0, 0)),
        out_shape=jax.ShapeDtypeStruct((B, 1, T), jnp.float32),
    )(hidden, w0, w1, w2, cb2, lw2, lb2, mask3)[:, 0, :]

    target_length = (target_label != -1).astype(jnp.float32).sum(-1)
    token_num = araw.sum(-1)
    scale = (target_length / token_num).reshape(1, B)

    arawT = araw.T
    alphasT, firesT, uT, vT, kAT, nfires = pl.pallas_call(
        _scan_kernel,
        out_shape=(
            jax.ShapeDtypeStruct((T, B), jnp.float32),
            jax.ShapeDtypeStruct((T, B), jnp.float32),
            jax.ShapeDtypeStruct((T, B), jnp.float32),
            jax.ShapeDtypeStruct((T, B), jnp.float32),
            jax.ShapeDtypeStruct((T, B), jnp.int32),
            jax.ShapeDtypeStruct((1, B), jnp.int32),
        ),
    )(arawT, scale)

    u3 = uT.T.reshape(B, 1, T)
    v3 = vT.T.reshape(B, 1, T)
    kA3 = kAT.T.reshape(B, 1, T)
    rowmask = (jnp.arange(L, dtype=jnp.int32)[None, :]
               < nfires[0][:, None]).astype(jnp.float32).reshape(B, 1, L)

    TT = 512
    NT = T // TT
    embeds = pl.pallas_call(
        _embed_kernel,
        grid=(B, NT),
        in_specs=[
            pl.BlockSpec((1, TT, D), lambda b, nt: (b, nt, 0)),
            pl.BlockSpec((1, 1, TT), lambda b, nt: (b, 0, nt)),
            pl.BlockSpec((1, 1, TT), lambda b, nt: (b, 0, nt)),
            pl.BlockSpec((1, 1, TT), lambda b, nt: (b, 0, nt)),
            pl.BlockSpec((1, 1, L), lambda b, nt: (b, 0, 0)),
        ],
        out_specs=pl.BlockSpec((1, L, D), lambda b, nt: (b, 0, 0)),
        out_shape=jax.ShapeDtypeStruct((B, L, D), jnp.float32),
    )(hidden, u3, v3, kA3, rowmask)

    return embeds, token_num, alphasT.T, firesT.T
